# baseline (speedup 1.0000x reference)
import functools

import numpy as np

import jax
import jax.numpy as jnp
from jax import lax
from jax.experimental import pallas as pl
from jax.experimental.pallas import tpu as pltpu
from jax.experimental.pallas import tpu_sc as plsc

N = 10000
E = 320000
G = 64
W = 128

EROW = 128
EPAD = 327680
NROWS = EPAD // EROW
NSUB = 16
NPAD = 10008
ZCH = 632
ZCH_LAST = N - 15 * ZCH

_sc_cache = {}

A_SUB = 32
B_SUB = 128


def _sc_agg():
    if "k" in _sc_cache:
        return _sc_cache["k"]

    mesh = plsc.VectorSubcoreMesh(core_axis_name="c", subcore_axis_name="s")
    slab = max(A_SUB, B_SUB) // 2

    @functools.partial(
        pl.kernel,
        out_type=jax.ShapeDtypeStruct((2, N, W), jnp.float32),
        mesh=mesh,
        scratch_types=(
            pltpu.VMEM((slab, EROW), jnp.int32),
            pltpu.VMEM((slab, EROW), jnp.int32),
            (pltpu.VMEM((EROW, W), jnp.float32),
             pltpu.VMEM((EROW, W), jnp.float32)),
            pltpu.VMEM_SHARED((NPAD, W), jnp.float32),
            (pltpu.SemaphoreType.DMA, pltpu.SemaphoreType.DMA),
        ),
    )
    def k(h, src2, dst2, zz, out, src_v, dst_v, bufs, shared, sems):
        cid = lax.axis_index("c")
        sid = lax.axis_index("s")
        rows_sub = jnp.where(cid == 0, A_SUB, B_SUB)
        r0 = jnp.where(cid == 0, sid * A_SUB, 16 * A_SUB + sid * B_SUB)
        z0 = sid * ZCH

        @pl.when(sid < 15)
        def _():
            pltpu.sync_copy(zz.at[pl.ds(z0, ZCH)], shared.at[pl.ds(z0, ZCH)])

        @pl.when(sid == 15)
        def _():
            pltpu.sync_copy(zz.at[pl.ds(15 * ZCH, ZCH_LAST)],
                            shared.at[pl.ds(15 * ZCH, ZCH_LAST)])

        plsc.subcore_barrier()

        rows0, rows1 = bufs
        sem0, sem1 = sems
        half = rows_sub // 2
        last = half - 1
        for hh in range(2):
            base = pl.multiple_of(r0 + hh * half, 8)
            pltpu.sync_copy(src2.at[pl.ds(base, slab)], src_v)
            pltpu.sync_copy(dst2.at[pl.ds(base, slab)], dst_v)
            pltpu.async_copy(h.at[src_v.at[0]], rows0, sem0)

            def body(jj, carry):
                j = jj * 2
                pltpu.make_async_copy(h.at[src_v.at[0]], rows0, sem0).wait()
                pltpu.async_copy(h.at[src_v.at[j + 1]], rows1, sem1)
                pltpu.sync_copy(rows0, shared.at[dst_v.at[j]], add=True)
                pltpu.make_async_copy(h.at[src_v.at[0]], rows1, sem1).wait()
                pltpu.async_copy(h.at[src_v.at[jnp.minimum(j + 2, last)]],
                                 rows0, sem0)
                pltpu.sync_copy(rows1, shared.at[dst_v.at[j + 1]], add=True)
                return carry

            lax.fori_loop(0, half // 2, body, 0)
            pltpu.make_async_copy(h.at[src_v.at[0]], rows0, sem0).wait()
        plsc.subcore_barrier()

        @pl.when(sid < 15)
        def _():
            pltpu.sync_copy(shared.at[pl.ds(z0, ZCH)],
                            out.at[cid, pl.ds(z0, ZCH)])

        @pl.when(sid == 15)
        def _():
            pltpu.sync_copy(shared.at[pl.ds(15 * ZCH, ZCH_LAST)],
                            out.at[cid, pl.ds(15 * ZCH, ZCH_LAST)])

    _sc_cache["k"] = k
    return k


BLK = 1000


def _ln_relu(m, gg, bee):
    mu = jnp.mean(m, axis=-1, keepdims=True)
    var = jnp.mean((m - mu) ** 2, axis=-1, keepdims=True)
    return jnp.maximum((m - mu) * lax.rsqrt(var + 1e-5) * gg + bee, 0.0)


def _dot(a, b):
    return jnp.dot(a, b, preferred_element_type=jnp.float32)


def _full(shape):
    return pl.BlockSpec(shape, lambda i: tuple(0 for _ in shape))


def _rowblk(w):
    return pl.BlockSpec((BLK, w), lambda i: (i, 0))


def _pblk():
    return pl.BlockSpec((2, BLK, 128), lambda i: (0, i, 0))


def _tc_layer1(p, x, w1rel, w1root, b1, g1, be1):
    def body(pr, xr, wrel, wro, b_r, g_r, be_r, oa, ob):
        m = (_dot(pr[0] + pr[1], wrel[...]) + _dot(xr[...], wro[...])
             + b_r[...])
        h = _ln_relu(m, g_r[...], be_r[...])
        oa[...] = h[:, :128]
        ob[...] = h[:, 128:]

    return pl.pallas_call(
        body,
        grid=(N // BLK,),
        in_specs=[_pblk(), _rowblk(128),
                  _full((128, 256)), _full((128, 256)),
                  _full((1, 256)), _full((1, 256)), _full((1, 256))],
        out_specs=[_rowblk(128), _rowblk(128)],
        out_shape=[jax.ShapeDtypeStruct((N, 128), jnp.float32)] * 2,
    )(p, x, w1rel, w1root, b1.reshape(1, -1), g1.reshape(1, -1),
      be1.reshape(1, -1))


def _tc_layer2(pa, pb, h1a, h1b, w2ra, w2rb, w2roa, w2rob,
               b2, g2, be2, w3rel, w3root, b3):
    def body(par, pbr, xa, xb, w2ra_r, w2rb_r, w2roa_r, w2rob_r,
             b2_r, g2_r, be2_r, w3rel_r, w3root_r, b3_r, yo, r3o):
        m = (_dot(par[0] + par[1], w2ra_r[...])
             + _dot(pbr[0] + pbr[1], w2rb_r[...])
             + _dot(xa[...], w2roa_r[...]) + _dot(xb[...], w2rob_r[...])
             + b2_r[...])
        h2 = _ln_relu(m, g2_r[...], be2_r[...])
        yo[...] = _dot(h2, w3rel_r[...])
        r3o[...] = _dot(h2, w3root_r[...]) + b3_r[...]

    return pl.pallas_call(
        body,
        grid=(N // BLK,),
        in_specs=[_pblk(), _pblk(), _rowblk(128), _rowblk(128)] + [
            _full((128, 256)), _full((128, 256)), _full((128, 256)),
            _full((128, 256)), _full((1, 256)), _full((1, 256)),
            _full((1, 256)), _full((256, 128)), _full((256, 128)),
            _full((1, 128))],
        out_specs=[_rowblk(128), _rowblk(128)],
        out_shape=[jax.ShapeDtypeStruct((N, 128), jnp.float32)] * 2,
    )(pa, pb, h1a, h1b, w2ra, w2rb, w2roa, w2rob,
      b2.reshape(1, -1), g2.reshape(1, -1), be2.reshape(1, -1),
      w3rel, w3root, b3.reshape(1, -1))


def _tc_layer3(q, r3, batch2d, g3, be3):
    def body(qr, r3_r, bt_r, g3_r, be3_r, out, sums, counts):
        i = pl.program_id(0)

        @pl.when(i == 0)
        def _():
            sums[...] = jnp.zeros_like(sums)
            counts[...] = jnp.zeros_like(counts)

        m = qr[0] + qr[1] + r3_r[...]
        t = _ln_relu(m, g3_r[...], be3_r[...])
        oh = (bt_r[...] == lax.broadcasted_iota(jnp.int32, (BLK, G), 1))
        oh = oh.astype(jnp.float32)
        sums[...] += lax.dot_general(oh, t, (((0,), (0,)), ((), ())),
                                     preferred_element_type=jnp.float32)
        counts[...] += lax.dot_general(
            oh, jnp.ones((BLK, 128), jnp.float32), (((0,), (0,)), ((), ())),
            preferred_element_type=jnp.float32)

        @pl.when(i == pl.num_programs(0) - 1)
        def _():
            out[...] = sums[...] / jnp.maximum(counts[...], 1.0)

    return pl.pallas_call(
        body,
        grid=(N // BLK,),
        in_specs=[_pblk(), _rowblk(128),
                  pl.BlockSpec((BLK, 1), lambda i: (i, 0)),
                  _full((1, 128)), _full((1, 128))],
        out_specs=pl.BlockSpec((G, 128), lambda i: (0, 0)),
        out_shape=jax.ShapeDtypeStruct((G, 128), jnp.float32),
        scratch_shapes=[pltpu.VMEM((G, 128), jnp.float32),
                        pltpu.VMEM((G, 128), jnp.float32)],
    )(q, r3, batch2d, g3.reshape(1, -1), be3.reshape(1, -1))


def kernel(x, edge_index, batch,
           W1_rel, W1_root, b1, g1, be1,
           W2_rel, W2_root, b2, g2, be2,
           W3_rel, W3_root, b3, g3, be3):
    src = edge_index[0]
    dst = edge_index[1]
    pad = EPAD - E
    src2 = jnp.concatenate([src, jnp.zeros((pad,), jnp.int32)]).reshape(
        NROWS, EROW)
    pad_dst = jnp.asarray(N + (np.arange(pad) % (NPAD - N)), dtype=jnp.int32)
    dst2 = jnp.concatenate([dst, pad_dst]).reshape(NROWS, EROW)
    z128 = jnp.zeros((N, 128), jnp.float32)

    p = _sc_agg()(x, src2, dst2, z128)
    h1a, h1b = _tc_layer1(p, x, W1_rel, W1_root, b1, g1, be1)
    pa = _sc_agg()(h1a, src2, dst2, z128)
    pb = _sc_agg()(h1b, src2, dst2, z128)
    y3, r3 = _tc_layer2(pa, pb, h1a, h1b,
                        W2_rel[:128], W2_rel[128:],
                        W2_root[:128], W2_root[128:],
                        b2, g2, be2, W3_rel, W3_root, b3)
    q = _sc_agg()(y3, src2, dst2, z128)
    return _tc_layer3(q, r3, batch.reshape(N, 1), g3, be3)


# speedup vs baseline: 1.1698x; 1.1698x over previous
import functools

import numpy as np

import jax
import jax.numpy as jnp
from jax import lax
from jax.experimental import pallas as pl
from jax.experimental.pallas import tpu as pltpu
from jax.experimental.pallas import tpu_sc as plsc

N = 10000
E = 320000
G = 64
W = 128

EROW = 128
EPAD = 327680
NROWS = EPAD // EROW
NSUB = 16
NPAD = 10008
ZCH = 632
ZCH_LAST = N - 15 * ZCH

_sc_cache = {}

A_SUB = 128
B_SUB = 32


def _sc_agg():
    if "k" in _sc_cache:
        return _sc_cache["k"]

    mesh = plsc.VectorSubcoreMesh(core_axis_name="c", subcore_axis_name="s")
    slab = max(A_SUB, B_SUB) // 2

    @functools.partial(
        pl.kernel,
        out_type=jax.ShapeDtypeStruct((2, N, W), jnp.float32),
        mesh=mesh,
        scratch_types=(
            pltpu.VMEM((slab, EROW), jnp.int32),
            pltpu.VMEM((slab, EROW), jnp.int32),
            (pltpu.VMEM((EROW, W), jnp.float32),
             pltpu.VMEM((EROW, W), jnp.float32)),
            pltpu.VMEM_SHARED((NPAD, W), jnp.float32),
            (pltpu.SemaphoreType.DMA, pltpu.SemaphoreType.DMA),
        ),
    )
    def k(h, src2, dst2, zz, out, src_v, dst_v, bufs, shared, sems):
        cid = lax.axis_index("c")
        sid = lax.axis_index("s")
        rows_sub = jnp.where(cid == 0, A_SUB, B_SUB)
        r0 = jnp.where(cid == 0, sid * A_SUB, 16 * A_SUB + sid * B_SUB)
        z0 = sid * ZCH

        @pl.when(sid < 15)
        def _():
            pltpu.sync_copy(zz.at[pl.ds(z0, ZCH)], shared.at[pl.ds(z0, ZCH)])

        @pl.when(sid == 15)
        def _():
            pltpu.sync_copy(zz.at[pl.ds(15 * ZCH, ZCH_LAST)],
                            shared.at[pl.ds(15 * ZCH, ZCH_LAST)])

        plsc.subcore_barrier()

        rows0, rows1 = bufs
        sem0, sem1 = sems
        half = rows_sub // 2
        last = half - 1
        for hh in range(2):
            start = r0 + hh * half
            base = pl.multiple_of(jnp.minimum(start, NROWS - slab), 8)
            off = start - base
            pltpu.sync_copy(src2.at[pl.ds(base, slab)], src_v)
            pltpu.sync_copy(dst2.at[pl.ds(base, slab)], dst_v)
            pltpu.async_copy(h.at[src_v.at[off]], rows0, sem0)

            def body(jj, carry):
                j = jj * 2
                pltpu.make_async_copy(h.at[src_v.at[0]], rows0, sem0).wait()
                pltpu.async_copy(h.at[src_v.at[off + j + 1]], rows1, sem1)
                pltpu.sync_copy(rows0, shared.at[dst_v.at[off + j]], add=True)
                pltpu.make_async_copy(h.at[src_v.at[0]], rows1, sem1).wait()
                pltpu.async_copy(
                    h.at[src_v.at[off + jnp.minimum(j + 2, last)]],
                    rows0, sem0)
                pltpu.sync_copy(rows1, shared.at[dst_v.at[off + j + 1]],
                                add=True)
                return carry

            lax.fori_loop(0, half // 2, body, 0)
            pltpu.make_async_copy(h.at[src_v.at[0]], rows0, sem0).wait()
        plsc.subcore_barrier()

        @pl.when(sid < 15)
        def _():
            pltpu.sync_copy(shared.at[pl.ds(z0, ZCH)],
                            out.at[cid, pl.ds(z0, ZCH)])

        @pl.when(sid == 15)
        def _():
            pltpu.sync_copy(shared.at[pl.ds(15 * ZCH, ZCH_LAST)],
                            out.at[cid, pl.ds(15 * ZCH, ZCH_LAST)])

    _sc_cache["k"] = k
    return k


BLK = 1000


def _ln_relu(m, gg, bee):
    mu = jnp.mean(m, axis=-1, keepdims=True)
    var = jnp.mean((m - mu) ** 2, axis=-1, keepdims=True)
    return jnp.maximum((m - mu) * lax.rsqrt(var + 1e-5) * gg + bee, 0.0)


def _dot(a, b):
    return jnp.dot(a, b, preferred_element_type=jnp.float32)


def _full(shape):
    return pl.BlockSpec(shape, lambda i: tuple(0 for _ in shape))


def _rowblk(w):
    return pl.BlockSpec((BLK, w), lambda i: (i, 0))


def _pblk():
    return pl.BlockSpec((2, BLK, 128), lambda i: (0, i, 0))


def _tc_layer1(p, x, w1rel, w1root, b1, g1, be1):
    def body(pr, xr, wrel, wro, b_r, g_r, be_r, oa, ob):
        m = (_dot(pr[0] + pr[1], wrel[...]) + _dot(xr[...], wro[...])
             + b_r[...])
        h = _ln_relu(m, g_r[...], be_r[...])
        oa[...] = h[:, :128]
        ob[...] = h[:, 128:]

    return pl.pallas_call(
        body,
        grid=(N // BLK,),
        in_specs=[_pblk(), _rowblk(128),
                  _full((128, 256)), _full((128, 256)),
                  _full((1, 256)), _full((1, 256)), _full((1, 256))],
        out_specs=[_rowblk(128), _rowblk(128)],
        out_shape=[jax.ShapeDtypeStruct((N, 128), jnp.float32)] * 2,
    )(p, x, w1rel, w1root, b1.reshape(1, -1), g1.reshape(1, -1),
      be1.reshape(1, -1))


def _tc_layer2(pa, pb, h1a, h1b, w2ra, w2rb, w2roa, w2rob,
               b2, g2, be2, w3rel, w3root, b3):
    def body(par, pbr, xa, xb, w2ra_r, w2rb_r, w2roa_r, w2rob_r,
             b2_r, g2_r, be2_r, w3rel_r, w3root_r, b3_r, yo, r3o):
        m = (_dot(par[0] + par[1], w2ra_r[...])
             + _dot(pbr[0] + pbr[1], w2rb_r[...])
             + _dot(xa[...], w2roa_r[...]) + _dot(xb[...], w2rob_r[...])
             + b2_r[...])
        h2 = _ln_relu(m, g2_r[...], be2_r[...])
        yo[...] = _dot(h2, w3rel_r[...])
        r3o[...] = _dot(h2, w3root_r[...]) + b3_r[...]

    return pl.pallas_call(
        body,
        grid=(N // BLK,),
        in_specs=[_pblk(), _pblk(), _rowblk(128), _rowblk(128)] + [
            _full((128, 256)), _full((128, 256)), _full((128, 256)),
            _full((128, 256)), _full((1, 256)), _full((1, 256)),
            _full((1, 256)), _full((256, 128)), _full((256, 128)),
            _full((1, 128))],
        out_specs=[_rowblk(128), _rowblk(128)],
        out_shape=[jax.ShapeDtypeStruct((N, 128), jnp.float32)] * 2,
    )(pa, pb, h1a, h1b, w2ra, w2rb, w2roa, w2rob,
      b2.reshape(1, -1), g2.reshape(1, -1), be2.reshape(1, -1),
      w3rel, w3root, b3.reshape(1, -1))


def _tc_layer3(q, r3, batch2d, g3, be3):
    def body(qr, r3_r, bt_r, g3_r, be3_r, out, sums, counts):
        i = pl.program_id(0)

        @pl.when(i == 0)
        def _():
            sums[...] = jnp.zeros_like(sums)
            counts[...] = jnp.zeros_like(counts)

        m = qr[0] + qr[1] + r3_r[...]
        t = _ln_relu(m, g3_r[...], be3_r[...])
        oh = (bt_r[...] == lax.broadcasted_iota(jnp.int32, (BLK, G), 1))
        oh = oh.astype(jnp.float32)
        sums[...] += lax.dot_general(oh, t, (((0,), (0,)), ((), ())),
                                     preferred_element_type=jnp.float32)
        counts[...] += lax.dot_general(
            oh, jnp.ones((BLK, 128), jnp.float32), (((0,), (0,)), ((), ())),
            preferred_element_type=jnp.float32)

        @pl.when(i == pl.num_programs(0) - 1)
        def _():
            out[...] = sums[...] / jnp.maximum(counts[...], 1.0)

    return pl.pallas_call(
        body,
        grid=(N // BLK,),
        in_specs=[_pblk(), _rowblk(128),
                  pl.BlockSpec((BLK, 1), lambda i: (i, 0)),
                  _full((1, 128)), _full((1, 128))],
        out_specs=pl.BlockSpec((G, 128), lambda i: (0, 0)),
        out_shape=jax.ShapeDtypeStruct((G, 128), jnp.float32),
        scratch_shapes=[pltpu.VMEM((G, 128), jnp.float32),
                        pltpu.VMEM((G, 128), jnp.float32)],
    )(q, r3, batch2d, g3.reshape(1, -1), be3.reshape(1, -1))


def kernel(x, edge_index, batch,
           W1_rel, W1_root, b1, g1, be1,
           W2_rel, W2_root, b2, g2, be2,
           W3_rel, W3_root, b3, g3, be3):
    src = edge_index[0]
    dst = edge_index[1]
    pad = EPAD - E
    src2 = jnp.concatenate([src, jnp.zeros((pad,), jnp.int32)]).reshape(
        NROWS, EROW)
    pad_dst = jnp.asarray(N + (np.arange(pad) % (NPAD - N)), dtype=jnp.int32)
    dst2 = jnp.concatenate([dst, pad_dst]).reshape(NROWS, EROW)
    z128 = jnp.zeros((N, 128), jnp.float32)

    p = _sc_agg()(x, src2, dst2, z128)
    h1a, h1b = _tc_layer1(p, x, W1_rel, W1_root, b1, g1, be1)
    pa = _sc_agg()(h1a, src2, dst2, z128)
    pb = _sc_agg()(h1b, src2, dst2, z128)
    y3, r3 = _tc_layer2(pa, pb, h1a, h1b,
                        W2_rel[:128], W2_rel[128:],
                        W2_root[:128], W2_root[128:],
                        b2, g2, be2, W3_rel, W3_root, b3)
    q = _sc_agg()(y3, src2, dst2, z128)
    return _tc_layer3(q, r3, batch.reshape(N, 1), g3, be3)


# speedup vs baseline: 3.5594x; 3.0426x over previous
import functools

import jax
import jax.numpy as jnp
from jax import lax
from jax.experimental import pallas as pl
from jax.experimental.pallas import tpu as pltpu
from jax.experimental.pallas import tpu_sc as plsc

N = 10000
E = 320000
G = 64
W = 128

EROW = 128
NROWS = E // EROW
NROWS_P = 2504
NSUB = 16
SLAB = 48
ZCH = 632
ZCH_LAST = N - 15 * ZCH

_sc_cache = {}


def _sc_agg():
    if "k" in _sc_cache:
        return _sc_cache["k"]

    mesh = plsc.VectorSubcoreMesh(core_axis_name="c", subcore_axis_name="s")

    @functools.partial(
        pl.kernel,
        out_type=jax.ShapeDtypeStruct((2, N, W), jnp.float32),
        mesh=mesh,
        scratch_types=(
            pltpu.VMEM((SLAB, EROW), jnp.int32),
            pltpu.VMEM((SLAB, EROW), jnp.int32),
            (pltpu.VMEM((EROW, W), jnp.float32),
             pltpu.VMEM((EROW, W), jnp.float32)),
            pltpu.VMEM_SHARED((N, W), jnp.float32),
            (pltpu.SemaphoreType.DMA, pltpu.SemaphoreType.DMA),
        ),
    )
    def k(h, src2, dst2, zz, out, src_v, dst_v, bufs, shared, sems):
        cid = lax.axis_index("c")
        sid = lax.axis_index("s")
        w = cid * NSUB + sid
        rows_w = jnp.where(w < 2, 80, 78)
        r0 = jnp.where(w < 2, 80 * w, 160 + 78 * (w - 2))
        z0 = sid * ZCH

        @pl.when(sid < 15)
        def _():
            pltpu.sync_copy(zz.at[pl.ds(z0, ZCH)], shared.at[pl.ds(z0, ZCH)])

        @pl.when(sid == 15)
        def _():
            pltpu.sync_copy(zz.at[pl.ds(15 * ZCH, ZCH_LAST)],
                            shared.at[pl.ds(15 * ZCH, ZCH_LAST)])

        plsc.subcore_barrier()

        rows0, rows1 = bufs
        sem0, sem1 = sems
        for win in range(2):
            start = r0 + win * 40
            size = jnp.where(win == 0, 40, rows_w - 40)
            clamped = jnp.minimum(start, NROWS_P - SLAB)
            base = pl.multiple_of(clamped - clamped % 8, 8)
            off = start - base
            last = size - 1
            pltpu.sync_copy(src2.at[pl.ds(base, SLAB)], src_v)
            pltpu.sync_copy(dst2.at[pl.ds(base, SLAB)], dst_v)
            pltpu.async_copy(h.at[src_v.at[off]], rows0, sem0)

            def body(jj, carry):
                j = jj * 2
                pltpu.make_async_copy(h.at[src_v.at[0]], rows0, sem0).wait()
                pltpu.async_copy(h.at[src_v.at[off + j + 1]], rows1, sem1)
                pltpu.sync_copy(rows0, shared.at[dst_v.at[off + j]], add=True)
                pltpu.make_async_copy(h.at[src_v.at[0]], rows1, sem1).wait()
                pltpu.async_copy(
                    h.at[src_v.at[off + jnp.minimum(j + 2, last)]],
                    rows0, sem0)
                pltpu.sync_copy(rows1, shared.at[dst_v.at[off + j + 1]],
                                add=True)
                return carry

            lax.fori_loop(0, size // 2, body, 0)
            pltpu.make_async_copy(h.at[src_v.at[0]], rows0, sem0).wait()
        plsc.subcore_barrier()

        @pl.when(sid < 15)
        def _():
            pltpu.sync_copy(shared.at[pl.ds(z0, ZCH)],
                            out.at[cid, pl.ds(z0, ZCH)])

        @pl.when(sid == 15)
        def _():
            pltpu.sync_copy(shared.at[pl.ds(15 * ZCH, ZCH_LAST)],
                            out.at[cid, pl.ds(15 * ZCH, ZCH_LAST)])

    _sc_cache["k"] = k
    return k


BLK = 1000


def _ln_relu(m, gg, bee):
    mu = jnp.mean(m, axis=-1, keepdims=True)
    var = jnp.mean((m - mu) ** 2, axis=-1, keepdims=True)
    return jnp.maximum((m - mu) * lax.rsqrt(var + 1e-5) * gg + bee, 0.0)


def _dot(a, b):
    return jnp.dot(a, b, preferred_element_type=jnp.float32)


def _full(shape):
    return pl.BlockSpec(shape, lambda i: tuple(0 for _ in shape))


def _rowblk(w):
    return pl.BlockSpec((BLK, w), lambda i: (i, 0))


def _pblk():
    return pl.BlockSpec((2, BLK, 128), lambda i: (0, i, 0))


def _tc_layer1(p, x, w1rel, w1root, b1, g1, be1):
    def body(pr, xr, wrel, wro, b_r, g_r, be_r, oa, ob):
        m = (_dot(pr[0] + pr[1], wrel[...]) + _dot(xr[...], wro[...])
             + b_r[...])
        h = _ln_relu(m, g_r[...], be_r[...])
        oa[...] = h[:, :128]
        ob[...] = h[:, 128:]

    return pl.pallas_call(
        body,
        grid=(N // BLK,),
        in_specs=[_pblk(), _rowblk(128),
                  _full((128, 256)), _full((128, 256)),
                  _full((1, 256)), _full((1, 256)), _full((1, 256))],
        out_specs=[_rowblk(128), _rowblk(128)],
        out_shape=[jax.ShapeDtypeStruct((N, 128), jnp.float32)] * 2,
    )(p, x, w1rel, w1root, b1.reshape(1, -1), g1.reshape(1, -1),
      be1.reshape(1, -1))


def _tc_layer2(pa, pb, h1a, h1b, w2ra, w2rb, w2roa, w2rob,
               b2, g2, be2, w3rel, w3root, b3):
    def body(par, pbr, xa, xb, w2ra_r, w2rb_r, w2roa_r, w2rob_r,
             b2_r, g2_r, be2_r, w3rel_r, w3root_r, b3_r, yo, r3o):
        m = (_dot(par[0] + par[1], w2ra_r[...])
             + _dot(pbr[0] + pbr[1], w2rb_r[...])
             + _dot(xa[...], w2roa_r[...]) + _dot(xb[...], w2rob_r[...])
             + b2_r[...])
        h2 = _ln_relu(m, g2_r[...], be2_r[...])
        yo[...] = _dot(h2, w3rel_r[...])
        r3o[...] = _dot(h2, w3root_r[...]) + b3_r[...]

    return pl.pallas_call(
        body,
        grid=(N // BLK,),
        in_specs=[_pblk(), _pblk(), _rowblk(128), _rowblk(128)] + [
            _full((128, 256)), _full((128, 256)), _full((128, 256)),
            _full((128, 256)), _full((1, 256)), _full((1, 256)),
            _full((1, 256)), _full((256, 128)), _full((256, 128)),
            _full((1, 128))],
        out_specs=[_rowblk(128), _rowblk(128)],
        out_shape=[jax.ShapeDtypeStruct((N, 128), jnp.float32)] * 2,
    )(pa, pb, h1a, h1b, w2ra, w2rb, w2roa, w2rob,
      b2.reshape(1, -1), g2.reshape(1, -1), be2.reshape(1, -1),
      w3rel, w3root, b3.reshape(1, -1))


def _tc_layer3(q, r3, batch2d, g3, be3):
    def body(qr, r3_r, bt_r, g3_r, be3_r, out, sums, counts):
        i = pl.program_id(0)

        @pl.when(i == 0)
        def _():
            sums[...] = jnp.zeros_like(sums)
            counts[...] = jnp.zeros_like(counts)

        m = qr[0] + qr[1] + r3_r[...]
        t = _ln_relu(m, g3_r[...], be3_r[...])
        oh = (bt_r[...] == lax.broadcasted_iota(jnp.int32, (BLK, G), 1))
        oh = oh.astype(jnp.float32)
        sums[...] += lax.dot_general(oh, t, (((0,), (0,)), ((), ())),
                                     preferred_element_type=jnp.float32)
        counts[...] += lax.dot_general(
            oh, jnp.ones((BLK, 128), jnp.float32), (((0,), (0,)), ((), ())),
            preferred_element_type=jnp.float32)

        @pl.when(i == pl.num_programs(0) - 1)
        def _():
            out[...] = sums[...] / jnp.maximum(counts[...], 1.0)

    return pl.pallas_call(
        body,
        grid=(N // BLK,),
        in_specs=[_pblk(), _rowblk(128),
                  pl.BlockSpec((BLK, 1), lambda i: (i, 0)),
                  _full((1, 128)), _full((1, 128))],
        out_specs=pl.BlockSpec((G, 128), lambda i: (0, 0)),
        out_shape=jax.ShapeDtypeStruct((G, 128), jnp.float32),
        scratch_shapes=[pltpu.VMEM((G, 128), jnp.float32),
                        pltpu.VMEM((G, 128), jnp.float32)],
    )(q, r3, batch2d, g3.reshape(1, -1), be3.reshape(1, -1))


def kernel(x, edge_index, batch,
           W1_rel, W1_root, b1, g1, be1,
           W2_rel, W2_root, b2, g2, be2,
           W3_rel, W3_root, b3, g3, be3):
    padz = jnp.zeros(((NROWS_P - NROWS) * EROW,), jnp.int32)
    src2 = jnp.concatenate([edge_index[0], padz]).reshape(NROWS_P, EROW)
    dst2 = jnp.concatenate([edge_index[1], padz]).reshape(NROWS_P, EROW)
    z128 = jnp.zeros((N, 128), jnp.float32)

    p = _sc_agg()(x, src2, dst2, z128)
    h1a, h1b = _tc_layer1(p, x, W1_rel, W1_root, b1, g1, be1)
    pa = _sc_agg()(h1a, src2, dst2, z128)
    pb = _sc_agg()(h1b, src2, dst2, z128)
    y3, r3 = _tc_layer2(pa, pb, h1a, h1b,
                        W2_rel[:128], W2_rel[128:],
                        W2_root[:128], W2_root[128:],
                        b2, g2, be2, W3_rel, W3_root, b3)
    q = _sc_agg()(y3, src2, dst2, z128)
    return _tc_layer3(q, r3, batch.reshape(N, 1), g3, be3)
